# baseline (speedup 1.0000x reference)
import functools

import jax
import jax.numpy as jnp
from jax import lax
from jax.experimental import pallas as pl
from jax.experimental.pallas import tpu as pltpu
from jax.experimental.pallas import tpu_sc as plsc

_TEMP = 0.2
_NC = 2
_NS = 16


def _mask_apply(x, mask_nodes, mask_token):
    N, D = x.shape
    M = mask_nodes.shape[0]
    CPR = 200
    n_copy = -(-N // CPR)
    copy_per_tile = -(-n_copy // _NS)
    MCH = 128
    n_sc = -(-M // MCH)
    sc_per_tile = -(-n_sc // _NS)
    mesh = plsc.VectorSubcoreMesh(core_axis_name="c", subcore_axis_name="s")

    @functools.partial(
        pl.kernel,
        out_type=jax.ShapeDtypeStruct((N, D), jnp.float32),
        mesh=mesh,
        scratch_types=[
            pltpu.VMEM((CPR, D), jnp.float32),
            pltpu.VMEM((MCH, D), jnp.float32),
            pltpu.VMEM((MCH,), jnp.int32),
            pltpu.VMEM((1, D), jnp.float32),
        ],
    )
    def k(x_hbm, mi_hbm, tok_hbm, xm_hbm, buf_v, trows_v, idx_v, tok_v):
        cid = lax.axis_index("c")
        sid = lax.axis_index("s")
        on0 = cid == 0
        for t in range(copy_per_tile):
            q = sid + _NS * t

            @pl.when(jnp.logical_and(on0, q < n_copy))
            def _():
                off = q * CPR
                pltpu.sync_copy(x_hbm.at[pl.ds(off, CPR)], buf_v)
                pltpu.sync_copy(buf_v, xm_hbm.at[pl.ds(off, CPR)])

        plsc.subcore_barrier()

        @pl.when(on0)
        def _():
            pltpu.sync_copy(tok_hbm, tok_v)

            def fill(j, carry):
                for kk in range(D // 16):
                    trows_v[j, pl.ds(kk * 16, 16)] = tok_v[0, pl.ds(kk * 16, 16)]
                return carry

            lax.fori_loop(0, MCH, fill, 0)
            for t in range(sc_per_tile):
                g = sid + _NS * t

                @pl.when(g < n_sc)
                def _():
                    start = jnp.minimum(g * MCH, M - MCH)
                    pltpu.sync_copy(mi_hbm.at[pl.ds(start, MCH)], idx_v)
                    pltpu.sync_copy(trows_v, xm_hbm.at[idx_v])

    return k(x, mask_nodes, mask_token)


def _edge_segsum(h, srcp, dstp, zrows, n_pad):
    N, D = h.shape
    NCH, CH = dstp.shape[1], dstp.shape[2]
    EPP = NCH * CH
    RPT = (N // _NS) // 8 * 8
    TAIL = N - _NS * RPT
    mesh = plsc.VectorSubcoreMesh(core_axis_name="c", subcore_axis_name="s")

    @functools.partial(
        pl.kernel,
        out_type=jax.ShapeDtypeStruct((_NC * N, D), jnp.float32),
        mesh=mesh,
        scratch_types=[
            pltpu.VMEM_SHARED((N + n_pad, D), jnp.float32),
            pltpu.VMEM((NCH, CH), jnp.int32),
            pltpu.VMEM((CH,), jnp.int32),
            pltpu.VMEM((CH,), jnp.int32),
            pltpu.VMEM((CH, D), jnp.float32),
            pltpu.VMEM((CH, D), jnp.float32),
            pltpu.SemaphoreType.DMA,
            pltpu.SemaphoreType.DMA,
        ],
    )
    def k(h_hbm, src_hbm, dst_hbm, z_hbm, out_hbm, acc_sh, di_v, si_a, si_b,
          rows_a, rows_b, sem_a, sem_b):
        cid = lax.axis_index("c")
        sid = lax.axis_index("s")
        wid = sid * _NC + cid
        pltpu.sync_copy(dst_hbm.at[wid], di_v)
        pltpu.sync_copy(z_hbm, acc_sh.at[pl.ds(sid * RPT, RPT)])
        if TAIL:
            @pl.when(sid == 0)
            def _():
                pltpu.sync_copy(z_hbm.at[pl.ds(0, TAIL)],
                                acc_sh.at[pl.ds(_NS * RPT, TAIL)])
        e0 = wid * EPP
        pltpu.sync_copy(src_hbm.at[pl.ds(e0, CH)], si_a)
        pltpu.async_copy(h_hbm.at[si_a], rows_a, sem_a)
        plsc.subcore_barrier()

        def body(g, carry):
            i = g * 2
            pltpu.sync_copy(src_hbm.at[pl.ds(e0 + (i + 1) * CH, CH)], si_b)
            pltpu.async_copy(h_hbm.at[si_b], rows_b, sem_b)
            pltpu.make_async_copy(h_hbm.at[pl.ds(0, CH)], rows_a, sem_a).wait()
            pltpu.sync_copy(rows_a, acc_sh.at[di_v.at[i]], add=True)

            @pl.when(i + 2 < NCH)
            def _():
                pltpu.sync_copy(src_hbm.at[pl.ds(e0 + (i + 2) * CH, CH)], si_a)
                pltpu.async_copy(h_hbm.at[si_a], rows_a, sem_a)

            pltpu.make_async_copy(h_hbm.at[pl.ds(0, CH)], rows_b, sem_b).wait()
            pltpu.sync_copy(rows_b, acc_sh.at[di_v.at[i + 1]], add=True)
            return carry

        lax.fori_loop(0, NCH // 2, body, 0)
        plsc.subcore_barrier()
        pltpu.sync_copy(
            acc_sh.at[pl.ds(sid * RPT, RPT)],
            out_hbm.at[pl.ds(cid * N + sid * RPT, RPT)],
        )
        if TAIL:
            @pl.when(sid == 0)
            def _():
                pltpu.sync_copy(
                    acc_sh.at[pl.ds(_NS * RPT, TAIL)],
                    out_hbm.at[pl.ds(cid * N + _NS * RPT, TAIL)],
                )

    return k(h, srcp, dstp, zrows)


def _dense_layer(h, agg2, p, gid2):
    N, D = h.shape
    Bg = 16
    Hh = p["W1"].shape[0]

    def body(h_ref, a_ref, w1_ref, mg_ref, mb_ref, w2_ref, g_ref, b_ref, gid_ref,
             ho_ref, pool_ref):
        h_ = h_ref[...]
        h2 = h_ + a_ref[0:N] + a_ref[N:2 * N]
        y = lax.dot_general(h2, w1_ref[...], (((1,), (1,)), ((), ())),
                            preferred_element_type=jnp.float32)
        mu = jnp.mean(y, axis=0, keepdims=True)
        var = jnp.mean((y - mu) ** 2, axis=0, keepdims=True)
        y = (y - mu) * lax.rsqrt(var + 1e-5) * mg_ref[...] + mb_ref[...]
        y = jnp.maximum(y, 0.0)
        z = lax.dot_general(y, w2_ref[...], (((1,), (1,)), ((), ())),
                            preferred_element_type=jnp.float32)
        mu2 = jnp.mean(z, axis=0, keepdims=True)
        var2 = jnp.mean((z - mu2) ** 2, axis=0, keepdims=True)
        z = (z - mu2) * lax.rsqrt(var2 + 1e-5) * g_ref[...] + b_ref[...]
        hn = jnp.maximum(z, 0.0)
        ho_ref[...] = hn
        oh = (gid_ref[...] == lax.broadcasted_iota(jnp.int32, (1, Bg), 1)).astype(
            jnp.float32)
        pool_ref[...] = lax.dot_general(oh, hn, (((0,), (0,)), ((), ())),
                                        preferred_element_type=jnp.float32)

    return pl.pallas_call(
        body,
        out_shape=(
            jax.ShapeDtypeStruct((N, p["W2"].shape[0]), jnp.float32),
            jax.ShapeDtypeStruct((Bg, p["W2"].shape[0]), jnp.float32),
        ),
    )(h, agg2, p["W1"], p["mbn_g"].reshape(1, Hh), p["mbn_b"].reshape(1, Hh),
      p["W2"], p["bn_g"].reshape(1, -1), p["bn_b"].reshape(1, -1), gid2)


def _head(ch, gh, pp):
    def body(ch_ref, gh_ref, w1_ref, b1_ref, w2_ref, b2_ref, out_ref):
        def proj(z):
            z1 = lax.dot_general(z, w1_ref[...], (((1,), (1,)), ((), ())),
                                 preferred_element_type=jnp.float32) + b1_ref[...]
            z1 = jnp.maximum(z1, 0.0)
            return lax.dot_general(z1, w2_ref[...], (((1,), (1,)), ((), ())),
                                   preferred_element_type=jnp.float32) + b2_ref[...]

        c_h = proj(ch_ref[...])
        c_m = proj(gh_ref[...])
        na = jnp.sqrt(jnp.sum(c_h * c_h, axis=1, keepdims=True))
        nb = jnp.sqrt(jnp.sum(c_m * c_m, axis=1, keepdims=True))
        outer = lax.dot_general(na, nb, (((1,), (1,)), ((), ())),
                                preferred_element_type=jnp.float32)
        sim = jnp.exp(
            lax.dot_general(c_h, c_m, (((1,), (1,)), ((), ())),
                            preferred_element_type=jnp.float32) / outer / _TEMP)
        Bg = sim.shape[0]
        eye = (lax.broadcasted_iota(jnp.int32, (Bg, Bg), 0)
               == lax.broadcasted_iota(jnp.int32, (Bg, Bg), 1)).astype(jnp.float32)
        pos = jnp.sum(sim * eye, axis=1, keepdims=True)
        tot = jnp.sum(sim, axis=1, keepdims=True)
        lvec = jnp.log(pos / (tot - pos))
        out_ref[...] = -jnp.mean(lvec) * jnp.ones((1, 1), jnp.float32)

    return pl.pallas_call(
        body,
        out_shape=jax.ShapeDtypeStruct((1, 1), jnp.float32),
    )(ch, gh, pp["W1"], pp["b1"].reshape(1, -1), pp["W2"], pp["b2"].reshape(1, -1))


def kernel(x, edge_index, graph_ids, mask_nodes, enc_params, con_params,
           proj_params, mask_token):
    N, D = x.shape
    src = edge_index[0].astype(jnp.int32)
    dst = edge_index[1].astype(jnp.int32)
    mask_nodes = mask_nodes.astype(jnp.int32)
    gid2 = graph_ids.astype(jnp.int32).reshape(N, 1)
    zrows = jnp.zeros(((N // _NS) // 8 * 8, D), jnp.float32)

    E = src.shape[0]
    NW = _NC * _NS
    EP = E // NW
    CH = 128
    NCH = -(-EP // CH)
    NCH += NCH % 2
    n_pad = NCH * CH - EP
    src_r = src.reshape(NW, EP)
    dst_r = dst.reshape(NW, EP)
    if n_pad:
        pad_s = jnp.zeros((NW, n_pad), jnp.int32)
        pad_d = jnp.broadcast_to(N + jnp.arange(n_pad, dtype=jnp.int32),
                                 (NW, n_pad))
        src_r = jnp.concatenate([src_r, pad_s], axis=1)
        dst_r = jnp.concatenate([dst_r, pad_d], axis=1)
    srcp = src_r.reshape(NW * NCH * CH)
    dstp = dst_r.reshape(NW, NCH, CH)

    xm = _mask_apply(x, mask_nodes, mask_token)

    def encoder(h0, params):
        h = h0
        pools = []
        for p in params:
            agg2 = _edge_segsum(h, srcp, dstp, zrows, n_pad)
            h, pool = _dense_layer(h, agg2, p, gid2)
            pools.append(pool)
        return h, jnp.concatenate(pools, axis=1)

    _, gh = encoder(xm, enc_params)
    _, ch = encoder(x, con_params)
    out = _head(ch, gh, proj_params)
    return out[0, 0]


# speedup vs baseline: 1.0061x; 1.0061x over previous
import functools

import jax
import jax.numpy as jnp
from jax import lax
from jax.experimental import pallas as pl
from jax.experimental.pallas import tpu as pltpu
from jax.experimental.pallas import tpu_sc as plsc

_TEMP = 0.2
_NC = 2
_NS = 16


def _mask_apply(x, mask_nodes, mask_token):
    N, D = x.shape
    M = mask_nodes.shape[0]
    CPR = 200
    n_copy = -(-N // CPR)
    copy_per_tile = -(-n_copy // _NS)
    MCH = 128
    n_sc = -(-M // MCH)
    sc_per_tile = -(-n_sc // _NS)
    mesh = plsc.VectorSubcoreMesh(core_axis_name="c", subcore_axis_name="s")

    @functools.partial(
        pl.kernel,
        out_type=jax.ShapeDtypeStruct((N, D), jnp.float32),
        mesh=mesh,
        scratch_types=[
            pltpu.VMEM((CPR, D), jnp.float32),
            pltpu.VMEM((MCH, D), jnp.float32),
            pltpu.VMEM((MCH,), jnp.int32),
            pltpu.VMEM((1, D), jnp.float32),
        ],
    )
    def k(x_hbm, mi_hbm, tok_hbm, xm_hbm, buf_v, trows_v, idx_v, tok_v):
        cid = lax.axis_index("c")
        sid = lax.axis_index("s")
        on0 = cid == 0
        for t in range(copy_per_tile):
            q = sid + _NS * t

            @pl.when(jnp.logical_and(on0, q < n_copy))
            def _():
                off = q * CPR
                pltpu.sync_copy(x_hbm.at[pl.ds(off, CPR)], buf_v)
                pltpu.sync_copy(buf_v, xm_hbm.at[pl.ds(off, CPR)])

        plsc.subcore_barrier()

        @pl.when(on0)
        def _():
            pltpu.sync_copy(tok_hbm, tok_v)

            def fill(j, carry):
                for kk in range(D // 16):
                    trows_v[j, pl.ds(kk * 16, 16)] = tok_v[0, pl.ds(kk * 16, 16)]
                return carry

            lax.fori_loop(0, MCH, fill, 0)
            for t in range(sc_per_tile):
                g = sid + _NS * t

                @pl.when(g < n_sc)
                def _():
                    start = jnp.minimum(g * MCH, M - MCH)
                    pltpu.sync_copy(mi_hbm.at[pl.ds(start, MCH)], idx_v)
                    pltpu.sync_copy(trows_v, xm_hbm.at[idx_v])

    return k(x, mask_nodes, mask_token)


def _edge_segsum(h, srcp, dstp, zrows, n_pad, n_ch):
    N, D = h.shape
    Dh = D // 2
    CH = 128
    EPP = n_ch * CH
    RPT = (N // _NS) // 8 * 8
    TAIL = N - _NS * RPT
    h2v = h.reshape(2 * N, Dh)
    mesh = plsc.VectorSubcoreMesh(core_axis_name="c", subcore_axis_name="s")

    @functools.partial(
        pl.kernel,
        out_type=jax.ShapeDtypeStruct((_NC, N, Dh), jnp.float32),
        mesh=mesh,
        compiler_params=pltpu.CompilerParams(use_tc_tiling_on_sc=False),
        scratch_types=[
            pltpu.VMEM_SHARED((N + n_pad, Dh), jnp.float32),
            pltpu.VMEM((CH,), jnp.int32),
            pltpu.VMEM((CH,), jnp.int32),
            pltpu.VMEM((CH,), jnp.int32),
            pltpu.VMEM((CH, Dh), jnp.float32),
            pltpu.SemaphoreType.DMA,
        ],
    )
    def k(h_hbm, src_hbm, dst_hbm, z_hbm, out_hbm, acc_sh, si_v, sg_v, di_v,
          rows_v, sem):
        cid = lax.axis_index("c")
        sid = lax.axis_index("s")
        pltpu.sync_copy(z_hbm, acc_sh.at[pl.ds(sid * RPT, RPT)])
        if TAIL:
            @pl.when(sid == 0)
            def _():
                pltpu.sync_copy(z_hbm.at[pl.ds(0, TAIL)],
                                acc_sh.at[pl.ds(_NS * RPT, TAIL)])
        plsc.subcore_barrier()
        e0 = sid * EPP

        def body(i, carry):
            e = e0 + i * CH
            pltpu.sync_copy(src_hbm.at[pl.ds(e, CH)], si_v)
            for kk in range(CH // 16):
                s = si_v[pl.ds(kk * 16, 16)]
                sg_v[pl.ds(kk * 16, 16)] = s + s + cid
            pltpu.sync_copy(dst_hbm.at[pl.ds(e, CH)], di_v)
            pltpu.async_copy(h_hbm.at[sg_v], rows_v, sem).wait()
            pltpu.sync_copy(rows_v, acc_sh.at[di_v], add=True)
            return carry

        lax.fori_loop(0, n_ch, body, 0)
        plsc.subcore_barrier()
        pltpu.sync_copy(
            acc_sh.at[pl.ds(sid * RPT, RPT)],
            out_hbm.at[cid, pl.ds(sid * RPT, RPT)],
        )
        if TAIL:
            @pl.when(sid == 0)
            def _():
                pltpu.sync_copy(
                    acc_sh.at[pl.ds(_NS * RPT, TAIL)],
                    out_hbm.at[cid, pl.ds(_NS * RPT, TAIL)],
                )

    return k(h2v, srcp, dstp, zrows)


def _dense_layer(h, agg2, p, gid2):
    N, D = h.shape
    Bg = 16
    Hh = p["W1"].shape[0]

    def body(h_ref, a_ref, w1_ref, mg_ref, mb_ref, w2_ref, g_ref, b_ref, gid_ref,
             ho_ref, pool_ref):
        h_ = h_ref[...]
        h2 = h_ + jnp.concatenate([a_ref[0], a_ref[1]], axis=1)
        y = lax.dot_general(h2, w1_ref[...], (((1,), (1,)), ((), ())),
                            preferred_element_type=jnp.float32)
        mu = jnp.mean(y, axis=0, keepdims=True)
        var = jnp.mean((y - mu) ** 2, axis=0, keepdims=True)
        y = (y - mu) * lax.rsqrt(var + 1e-5) * mg_ref[...] + mb_ref[...]
        y = jnp.maximum(y, 0.0)
        z = lax.dot_general(y, w2_ref[...], (((1,), (1,)), ((), ())),
                            preferred_element_type=jnp.float32)
        mu2 = jnp.mean(z, axis=0, keepdims=True)
        var2 = jnp.mean((z - mu2) ** 2, axis=0, keepdims=True)
        z = (z - mu2) * lax.rsqrt(var2 + 1e-5) * g_ref[...] + b_ref[...]
        hn = jnp.maximum(z, 0.0)
        ho_ref[...] = hn
        oh = (gid_ref[...] == lax.broadcasted_iota(jnp.int32, (1, Bg), 1)).astype(
            jnp.float32)
        pool_ref[...] = lax.dot_general(oh, hn, (((0,), (0,)), ((), ())),
                                        preferred_element_type=jnp.float32)

    return pl.pallas_call(
        body,
        out_shape=(
            jax.ShapeDtypeStruct((N, p["W2"].shape[0]), jnp.float32),
            jax.ShapeDtypeStruct((Bg, p["W2"].shape[0]), jnp.float32),
        ),
    )(h, agg2, p["W1"], p["mbn_g"].reshape(1, Hh), p["mbn_b"].reshape(1, Hh),
      p["W2"], p["bn_g"].reshape(1, -1), p["bn_b"].reshape(1, -1), gid2)


def _head(ch, gh, pp):
    def body(ch_ref, gh_ref, w1_ref, b1_ref, w2_ref, b2_ref, out_ref):
        def proj(z):
            z1 = lax.dot_general(z, w1_ref[...], (((1,), (1,)), ((), ())),
                                 preferred_element_type=jnp.float32) + b1_ref[...]
            z1 = jnp.maximum(z1, 0.0)
            return lax.dot_general(z1, w2_ref[...], (((1,), (1,)), ((), ())),
                                   preferred_element_type=jnp.float32) + b2_ref[...]

        c_h = proj(ch_ref[...])
        c_m = proj(gh_ref[...])
        na = jnp.sqrt(jnp.sum(c_h * c_h, axis=1, keepdims=True))
        nb = jnp.sqrt(jnp.sum(c_m * c_m, axis=1, keepdims=True))
        outer = lax.dot_general(na, nb, (((1,), (1,)), ((), ())),
                                preferred_element_type=jnp.float32)
        sim = jnp.exp(
            lax.dot_general(c_h, c_m, (((1,), (1,)), ((), ())),
                            preferred_element_type=jnp.float32) / outer / _TEMP)
        Bg = sim.shape[0]
        eye = (lax.broadcasted_iota(jnp.int32, (Bg, Bg), 0)
               == lax.broadcasted_iota(jnp.int32, (Bg, Bg), 1)).astype(jnp.float32)
        pos = jnp.sum(sim * eye, axis=1, keepdims=True)
        tot = jnp.sum(sim, axis=1, keepdims=True)
        lvec = jnp.log(pos / (tot - pos))
        out_ref[...] = -jnp.mean(lvec) * jnp.ones((1, 1), jnp.float32)

    return pl.pallas_call(
        body,
        out_shape=jax.ShapeDtypeStruct((1, 1), jnp.float32),
    )(ch, gh, pp["W1"], pp["b1"].reshape(1, -1), pp["W2"], pp["b2"].reshape(1, -1))


def kernel(x, edge_index, graph_ids, mask_nodes, enc_params, con_params,
           proj_params, mask_token):
    N, D = x.shape
    src = edge_index[0].astype(jnp.int32)
    dst = edge_index[1].astype(jnp.int32)
    mask_nodes = mask_nodes.astype(jnp.int32)
    gid2 = graph_ids.astype(jnp.int32).reshape(N, 1)
    zrows = jnp.zeros(((N // _NS) // 8 * 8, D // 2), jnp.float32)

    E = src.shape[0]
    EP = E // _NS
    CH = 128
    NCH = -(-EP // CH)
    n_pad = NCH * CH - EP
    src_r = src.reshape(_NS, EP)
    dst_r = dst.reshape(_NS, EP)
    if n_pad:
        pad_s = jnp.zeros((_NS, n_pad), jnp.int32)
        pad_d = jnp.broadcast_to(N + jnp.arange(n_pad, dtype=jnp.int32),
                                 (_NS, n_pad))
        src_r = jnp.concatenate([src_r, pad_s], axis=1)
        dst_r = jnp.concatenate([dst_r, pad_d], axis=1)
    srcp = src_r.reshape(_NS * NCH * CH)
    dstp = dst_r.reshape(_NS * NCH * CH)
    n_pad_rows = max(n_pad, 8)

    xm = _mask_apply(x, mask_nodes, mask_token)

    def encoder(h0, params):
        h = h0
        pools = []
        for p in params:
            agg2 = _edge_segsum(h, srcp, dstp, zrows, n_pad_rows, NCH)
            h, pool = _dense_layer(h, agg2, p, gid2)
            pools.append(pool)
        return h, jnp.concatenate(pools, axis=1)

    _, gh = encoder(xm, enc_params)
    _, ch = encoder(x, con_params)
    out = _head(ch, gh, proj_params)
    return out[0, 0]


# speedup vs baseline: 1.6049x; 1.5951x over previous
import functools

import jax
import jax.numpy as jnp
from jax import lax
from jax.experimental import pallas as pl
from jax.experimental.pallas import tpu as pltpu
from jax.experimental.pallas import tpu_sc as plsc

_TEMP = 0.2
_NC = 2
_NS = 16
_CH = 512


def _mask_apply(x, mask_nodes, mask_token):
    N, D = x.shape
    M = mask_nodes.shape[0]
    CPR = 200
    n_copy = -(-N // CPR)
    copy_per_tile = -(-n_copy // _NS)
    MCH = 128
    n_sc = -(-M // MCH)
    sc_per_tile = -(-n_sc // _NS)
    mesh = plsc.VectorSubcoreMesh(core_axis_name="c", subcore_axis_name="s")

    @functools.partial(
        pl.kernel,
        out_type=jax.ShapeDtypeStruct((N, D), jnp.float32),
        mesh=mesh,
        scratch_types=[
            pltpu.VMEM((CPR, D), jnp.float32),
            pltpu.VMEM((MCH, D), jnp.float32),
            pltpu.VMEM((MCH,), jnp.int32),
            pltpu.VMEM((1, D), jnp.float32),
        ],
    )
    def k(x_hbm, mi_hbm, tok_hbm, xm_hbm, buf_v, trows_v, idx_v, tok_v):
        cid = lax.axis_index("c")
        sid = lax.axis_index("s")
        on0 = cid == 0
        for t in range(copy_per_tile):
            q = sid + _NS * t

            @pl.when(jnp.logical_and(on0, q < n_copy))
            def _():
                off = q * CPR
                pltpu.sync_copy(x_hbm.at[pl.ds(off, CPR)], buf_v)
                pltpu.sync_copy(buf_v, xm_hbm.at[pl.ds(off, CPR)])

        plsc.subcore_barrier()

        @pl.when(on0)
        def _():
            pltpu.sync_copy(tok_hbm, tok_v)

            def fill(j, carry):
                for kk in range(D // 16):
                    trows_v[j, pl.ds(kk * 16, 16)] = tok_v[0, pl.ds(kk * 16, 16)]
                return carry

            lax.fori_loop(0, MCH, fill, 0)
            for t in range(sc_per_tile):
                g = sid + _NS * t

                @pl.when(g < n_sc)
                def _():
                    start = jnp.minimum(g * MCH, M - MCH)
                    pltpu.sync_copy(mi_hbm.at[pl.ds(start, MCH)], idx_v)
                    pltpu.sync_copy(trows_v, xm_hbm.at[idx_v])

    return k(x, mask_nodes, mask_token)


def _edge_segsum(h, srcp, dstp, zrows, n_pad, n_ch):
    N = h.shape[0]
    D = h.shape[1]
    CH = _CH
    EPP = n_ch * CH
    RPT = (N // _NS) // 8 * 8
    TAIL = N - _NS * RPT
    dt = h.dtype
    mesh = plsc.VectorSubcoreMesh(core_axis_name="c", subcore_axis_name="s")

    @functools.partial(
        pl.kernel,
        out_type=jax.ShapeDtypeStruct((_NC, N, D), dt),
        mesh=mesh,
        compiler_params=pltpu.CompilerParams(use_tc_tiling_on_sc=False),
        scratch_types=[
            pltpu.VMEM_SHARED((N + n_pad, D), dt),
            pltpu.VMEM((CH,), jnp.int32),
            pltpu.VMEM((CH,), jnp.int32),
            pltpu.VMEM((CH, D), dt),
            pltpu.SemaphoreType.DMA,
        ],
    )
    def k(h_hbm, src_hbm, dst_hbm, z_hbm, out_hbm, acc_sh, si_v, di_v,
          rows_v, sem):
        cid = lax.axis_index("c")
        sid = lax.axis_index("s")
        wid = sid * _NC + cid
        pltpu.sync_copy(z_hbm, acc_sh.at[pl.ds(sid * RPT, RPT)])
        if TAIL:
            @pl.when(sid == 0)
            def _():
                pltpu.sync_copy(z_hbm.at[pl.ds(0, TAIL)],
                                acc_sh.at[pl.ds(_NS * RPT, TAIL)])
        plsc.subcore_barrier()
        e0 = wid * EPP

        def body(i, carry):
            e = e0 + i * CH
            pltpu.sync_copy(src_hbm.at[pl.ds(e, CH)], si_v)
            pltpu.sync_copy(dst_hbm.at[pl.ds(e, CH)], di_v)
            pltpu.async_copy(h_hbm.at[si_v], rows_v, sem).wait()
            pltpu.sync_copy(rows_v, acc_sh.at[di_v], add=True)
            return carry

        lax.fori_loop(0, n_ch, body, 0)
        plsc.subcore_barrier()
        pltpu.sync_copy(
            acc_sh.at[pl.ds(sid * RPT, RPT)],
            out_hbm.at[cid, pl.ds(sid * RPT, RPT)],
        )
        if TAIL:
            @pl.when(sid == 0)
            def _():
                pltpu.sync_copy(
                    acc_sh.at[pl.ds(_NS * RPT, TAIL)],
                    out_hbm.at[cid, pl.ds(_NS * RPT, TAIL)],
                )

    return k(h, srcp, dstp, zrows)


def _cast_bf16(a):
    def body(a_ref, o_ref):
        o_ref[...] = a_ref[...].astype(jnp.bfloat16)

    return pl.pallas_call(
        body, out_shape=jax.ShapeDtypeStruct(a.shape, jnp.bfloat16))(a)


def _dense_layer(h, agg2, p, gid2):
    N, D = h.shape
    Bg = 16
    Hh = p["W1"].shape[0]

    def body(h_ref, a_ref, w1_ref, mg_ref, mb_ref, w2_ref, g_ref, b_ref, gid_ref,
             ho_ref, hobf_ref, pool_ref):
        h_ = h_ref[...]
        h2 = h_ + a_ref[0].astype(jnp.float32) + a_ref[1].astype(jnp.float32)
        y = lax.dot_general(h2, w1_ref[...], (((1,), (1,)), ((), ())),
                            preferred_element_type=jnp.float32)
        mu = jnp.mean(y, axis=0, keepdims=True)
        var = jnp.mean((y - mu) ** 2, axis=0, keepdims=True)
        y = (y - mu) * lax.rsqrt(var + 1e-5) * mg_ref[...] + mb_ref[...]
        y = jnp.maximum(y, 0.0)
        z = lax.dot_general(y, w2_ref[...], (((1,), (1,)), ((), ())),
                            preferred_element_type=jnp.float32)
        mu2 = jnp.mean(z, axis=0, keepdims=True)
        var2 = jnp.mean((z - mu2) ** 2, axis=0, keepdims=True)
        z = (z - mu2) * lax.rsqrt(var2 + 1e-5) * g_ref[...] + b_ref[...]
        hn = jnp.maximum(z, 0.0)
        ho_ref[...] = hn
        hobf_ref[...] = hn.astype(jnp.bfloat16)
        oh = (gid_ref[...] == lax.broadcasted_iota(jnp.int32, (1, Bg), 1)).astype(
            jnp.float32)
        pool_ref[...] = lax.dot_general(oh, hn, (((0,), (0,)), ((), ())),
                                        preferred_element_type=jnp.float32)

    return pl.pallas_call(
        body,
        out_shape=(
            jax.ShapeDtypeStruct((N, p["W2"].shape[0]), jnp.float32),
            jax.ShapeDtypeStruct((N, p["W2"].shape[0]), jnp.bfloat16),
            jax.ShapeDtypeStruct((Bg, p["W2"].shape[0]), jnp.float32),
        ),
    )(h, agg2, p["W1"], p["mbn_g"].reshape(1, Hh), p["mbn_b"].reshape(1, Hh),
      p["W2"], p["bn_g"].reshape(1, -1), p["bn_b"].reshape(1, -1), gid2)


def _head(ch, gh, pp):
    def body(ch_ref, gh_ref, w1_ref, b1_ref, w2_ref, b2_ref, out_ref):
        def proj(z):
            z1 = lax.dot_general(z, w1_ref[...], (((1,), (1,)), ((), ())),
                                 preferred_element_type=jnp.float32) + b1_ref[...]
            z1 = jnp.maximum(z1, 0.0)
            return lax.dot_general(z1, w2_ref[...], (((1,), (1,)), ((), ())),
                                   preferred_element_type=jnp.float32) + b2_ref[...]

        c_h = proj(ch_ref[...])
        c_m = proj(gh_ref[...])
        na = jnp.sqrt(jnp.sum(c_h * c_h, axis=1, keepdims=True))
        nb = jnp.sqrt(jnp.sum(c_m * c_m, axis=1, keepdims=True))
        outer = lax.dot_general(na, nb, (((1,), (1,)), ((), ())),
                                preferred_element_type=jnp.float32)
        sim = jnp.exp(
            lax.dot_general(c_h, c_m, (((1,), (1,)), ((), ())),
                            preferred_element_type=jnp.float32) / outer / _TEMP)
        Bg = sim.shape[0]
        eye = (lax.broadcasted_iota(jnp.int32, (Bg, Bg), 0)
               == lax.broadcasted_iota(jnp.int32, (Bg, Bg), 1)).astype(jnp.float32)
        pos = jnp.sum(sim * eye, axis=1, keepdims=True)
        tot = jnp.sum(sim, axis=1, keepdims=True)
        lvec = jnp.log(pos / (tot - pos))
        out_ref[...] = -jnp.mean(lvec) * jnp.ones((1, 1), jnp.float32)

    return pl.pallas_call(
        body,
        out_shape=jax.ShapeDtypeStruct((1, 1), jnp.float32),
    )(ch, gh, pp["W1"], pp["b1"].reshape(1, -1), pp["W2"], pp["b2"].reshape(1, -1))


def kernel(x, edge_index, graph_ids, mask_nodes, enc_params, con_params,
           proj_params, mask_token):
    N, D = x.shape
    src = edge_index[0].astype(jnp.int32)
    dst = edge_index[1].astype(jnp.int32)
    mask_nodes = mask_nodes.astype(jnp.int32)
    gid2 = graph_ids.astype(jnp.int32).reshape(N, 1)
    zrows = jnp.zeros(((N // _NS) // 8 * 8, D), jnp.bfloat16)

    E = src.shape[0]
    NW = _NC * _NS
    EP = E // NW
    CH = _CH
    NCH = -(-EP // CH)
    n_pad = NCH * CH - EP
    src_r = src.reshape(NW, EP)
    dst_r = dst.reshape(NW, EP)
    if n_pad:
        pad_s = jnp.zeros((NW, n_pad), jnp.int32)
        pad_d = jnp.broadcast_to(N + jnp.arange(n_pad, dtype=jnp.int32),
                                 (NW, n_pad))
        src_r = jnp.concatenate([src_r, pad_s], axis=1)
        dst_r = jnp.concatenate([dst_r, pad_d], axis=1)
    srcp = src_r.reshape(NW * NCH * CH)
    dstp = dst_r.reshape(NW * NCH * CH)
    n_pad_rows = max(n_pad, 8)

    xm = _mask_apply(x, mask_nodes, mask_token)

    def encoder(h0, params):
        h = h0
        hb = _cast_bf16(h0)
        pools = []
        for p in params:
            agg2 = _edge_segsum(hb, srcp, dstp, zrows, n_pad_rows, NCH)
            h, hb, pool = _dense_layer(h, agg2, p, gid2)
            pools.append(pool)
        return h, jnp.concatenate(pools, axis=1)

    _, gh = encoder(xm, enc_params)
    _, ch = encoder(x, con_params)
    out = _head(ch, gh, proj_params)
    return out[0, 0]


# speedup vs baseline: 1.8208x; 1.1345x over previous
import functools

import jax
import jax.numpy as jnp
from jax import lax
from jax.experimental import pallas as pl
from jax.experimental.pallas import tpu as pltpu
from jax.experimental.pallas import tpu_sc as plsc

_TEMP = 0.2
_NC = 2
_NS = 16
_CH = 512


def _mask_apply(x, mask_nodes, mask_token):
    N, D = x.shape
    M = mask_nodes.shape[0]
    CPR = 200
    n_copy = -(-N // CPR)
    copy_per_tile = -(-n_copy // _NS)
    MCH = 128
    n_sc = -(-M // MCH)
    sc_per_tile = -(-n_sc // _NS)
    mesh = plsc.VectorSubcoreMesh(core_axis_name="c", subcore_axis_name="s")

    @functools.partial(
        pl.kernel,
        out_type=jax.ShapeDtypeStruct((N, D), jnp.float32),
        mesh=mesh,
        scratch_types=[
            pltpu.VMEM((CPR, D), jnp.float32),
            pltpu.VMEM((MCH, D), jnp.float32),
            pltpu.VMEM((MCH,), jnp.int32),
            pltpu.VMEM((1, D), jnp.float32),
        ],
    )
    def k(x_hbm, mi_hbm, tok_hbm, xm_hbm, buf_v, trows_v, idx_v, tok_v):
        cid = lax.axis_index("c")
        sid = lax.axis_index("s")
        on0 = cid == 0
        for t in range(copy_per_tile):
            q = sid + _NS * t

            @pl.when(jnp.logical_and(on0, q < n_copy))
            def _():
                off = q * CPR
                pltpu.sync_copy(x_hbm.at[pl.ds(off, CPR)], buf_v)
                pltpu.sync_copy(buf_v, xm_hbm.at[pl.ds(off, CPR)])

        plsc.subcore_barrier()

        @pl.when(on0)
        def _():
            pltpu.sync_copy(tok_hbm, tok_v)

            def fill(j, carry):
                for kk in range(D // 16):
                    trows_v[j, pl.ds(kk * 16, 16)] = tok_v[0, pl.ds(kk * 16, 16)]
                return carry

            lax.fori_loop(0, MCH, fill, 0)
            for t in range(sc_per_tile):
                g = sid + _NS * t

                @pl.when(g < n_sc)
                def _():
                    start = jnp.minimum(g * MCH, M - MCH)
                    pltpu.sync_copy(mi_hbm.at[pl.ds(start, MCH)], idx_v)
                    pltpu.sync_copy(trows_v, xm_hbm.at[idx_v])

    return k(x, mask_nodes, mask_token)


def _edge_segsum(h, srcp, dstp, zrows, n_pad, n_ch):
    N = h.shape[0]
    D = h.shape[1]
    CH = _CH
    EPP = n_ch * CH
    RPT = (N // _NS) // 8 * 8
    TAIL = N - _NS * RPT
    dt = h.dtype
    mesh = plsc.VectorSubcoreMesh(core_axis_name="c", subcore_axis_name="s")

    @functools.partial(
        pl.kernel,
        out_type=jax.ShapeDtypeStruct((_NC, N, D), dt),
        mesh=mesh,
        compiler_params=pltpu.CompilerParams(use_tc_tiling_on_sc=False),
        scratch_types=[
            pltpu.VMEM_SHARED((N + n_pad, D), dt),
            pltpu.VMEM((CH,), jnp.int32),
            pltpu.VMEM((CH,), jnp.int32),
            pltpu.VMEM((CH,), jnp.int32),
            pltpu.VMEM((CH,), jnp.int32),
            pltpu.VMEM((CH, D), dt),
            pltpu.VMEM((CH, D), dt),
            pltpu.SemaphoreType.DMA,
            pltpu.SemaphoreType.DMA,
        ],
    )
    def k(h_hbm, src_hbm, dst_hbm, z_hbm, out_hbm, acc_sh, si_a, si_b,
          di_a, di_b, rows_a, rows_b, sem_a, sem_b):
        cid = lax.axis_index("c")
        sid = lax.axis_index("s")
        wid = sid * _NC + cid
        pltpu.sync_copy(z_hbm, acc_sh.at[pl.ds(sid * RPT, RPT)])
        if TAIL:
            @pl.when(sid == 0)
            def _():
                pltpu.sync_copy(z_hbm.at[pl.ds(0, TAIL)],
                                acc_sh.at[pl.ds(_NS * RPT, TAIL)])
        e0 = wid * EPP
        pltpu.sync_copy(src_hbm.at[pl.ds(e0, CH)], si_a)
        pltpu.async_copy(h_hbm.at[si_a], rows_a, sem_a)
        plsc.subcore_barrier()

        def body(g, carry):
            i = g * 2
            pltpu.sync_copy(src_hbm.at[pl.ds(e0 + (i + 1) * CH, CH)], si_b)
            pltpu.async_copy(h_hbm.at[si_b], rows_b, sem_b)
            pltpu.sync_copy(dst_hbm.at[pl.ds(e0 + i * CH, CH)], di_a)
            pltpu.make_async_copy(h_hbm.at[pl.ds(0, CH)], rows_a, sem_a).wait()
            pltpu.sync_copy(rows_a, acc_sh.at[di_a], add=True)

            @pl.when(i + 2 < n_ch)
            def _():
                pltpu.sync_copy(src_hbm.at[pl.ds(e0 + (i + 2) * CH, CH)], si_a)
                pltpu.async_copy(h_hbm.at[si_a], rows_a, sem_a)

            pltpu.sync_copy(dst_hbm.at[pl.ds(e0 + (i + 1) * CH, CH)], di_b)
            pltpu.make_async_copy(h_hbm.at[pl.ds(0, CH)], rows_b, sem_b).wait()
            pltpu.sync_copy(rows_b, acc_sh.at[di_b], add=True)
            return carry

        lax.fori_loop(0, n_ch // 2, body, 0)
        plsc.subcore_barrier()
        pltpu.sync_copy(
            acc_sh.at[pl.ds(sid * RPT, RPT)],
            out_hbm.at[cid, pl.ds(sid * RPT, RPT)],
        )
        if TAIL:
            @pl.when(sid == 0)
            def _():
                pltpu.sync_copy(
                    acc_sh.at[pl.ds(_NS * RPT, TAIL)],
                    out_hbm.at[cid, pl.ds(_NS * RPT, TAIL)],
                )

    return k(h, srcp, dstp, zrows)


def _cast_bf16(a):
    def body(a_ref, o_ref):
        o_ref[...] = a_ref[...].astype(jnp.bfloat16)

    return pl.pallas_call(
        body, out_shape=jax.ShapeDtypeStruct(a.shape, jnp.bfloat16))(a)


def _dense_layer(h, agg2, p, gid2):
    N, D = h.shape
    Bg = 16
    Hh = p["W1"].shape[0]

    def body(h_ref, a_ref, w1_ref, mg_ref, mb_ref, w2_ref, g_ref, b_ref, gid_ref,
             ho_ref, hobf_ref, pool_ref):
        h_ = h_ref[...]
        h2 = h_ + a_ref[0].astype(jnp.float32) + a_ref[1].astype(jnp.float32)
        y = lax.dot_general(h2, w1_ref[...], (((1,), (1,)), ((), ())),
                            preferred_element_type=jnp.float32)
        mu = jnp.mean(y, axis=0, keepdims=True)
        var = jnp.mean((y - mu) ** 2, axis=0, keepdims=True)
        y = (y - mu) * lax.rsqrt(var + 1e-5) * mg_ref[...] + mb_ref[...]
        y = jnp.maximum(y, 0.0)
        z = lax.dot_general(y, w2_ref[...], (((1,), (1,)), ((), ())),
                            preferred_element_type=jnp.float32)
        mu2 = jnp.mean(z, axis=0, keepdims=True)
        var2 = jnp.mean((z - mu2) ** 2, axis=0, keepdims=True)
        z = (z - mu2) * lax.rsqrt(var2 + 1e-5) * g_ref[...] + b_ref[...]
        hn = jnp.maximum(z, 0.0)
        ho_ref[...] = hn
        hobf_ref[...] = hn.astype(jnp.bfloat16)
        oh = (gid_ref[...] == lax.broadcasted_iota(jnp.int32, (1, Bg), 1)).astype(
            jnp.float32)
        pool_ref[...] = lax.dot_general(oh, hn, (((0,), (0,)), ((), ())),
                                        preferred_element_type=jnp.float32)

    return pl.pallas_call(
        body,
        out_shape=(
            jax.ShapeDtypeStruct((N, p["W2"].shape[0]), jnp.float32),
            jax.ShapeDtypeStruct((N, p["W2"].shape[0]), jnp.bfloat16),
            jax.ShapeDtypeStruct((Bg, p["W2"].shape[0]), jnp.float32),
        ),
    )(h, agg2, p["W1"], p["mbn_g"].reshape(1, Hh), p["mbn_b"].reshape(1, Hh),
      p["W2"], p["bn_g"].reshape(1, -1), p["bn_b"].reshape(1, -1), gid2)


def _head(ch, gh, pp):
    def body(ch_ref, gh_ref, w1_ref, b1_ref, w2_ref, b2_ref, out_ref):
        def proj(z):
            z1 = lax.dot_general(z, w1_ref[...], (((1,), (1,)), ((), ())),
                                 preferred_element_type=jnp.float32) + b1_ref[...]
            z1 = jnp.maximum(z1, 0.0)
            return lax.dot_general(z1, w2_ref[...], (((1,), (1,)), ((), ())),
                                   preferred_element_type=jnp.float32) + b2_ref[...]

        c_h = proj(ch_ref[...])
        c_m = proj(gh_ref[...])
        na = jnp.sqrt(jnp.sum(c_h * c_h, axis=1, keepdims=True))
        nb = jnp.sqrt(jnp.sum(c_m * c_m, axis=1, keepdims=True))
        outer = lax.dot_general(na, nb, (((1,), (1,)), ((), ())),
                                preferred_element_type=jnp.float32)
        sim = jnp.exp(
            lax.dot_general(c_h, c_m, (((1,), (1,)), ((), ())),
                            preferred_element_type=jnp.float32) / outer / _TEMP)
        Bg = sim.shape[0]
        eye = (lax.broadcasted_iota(jnp.int32, (Bg, Bg), 0)
               == lax.broadcasted_iota(jnp.int32, (Bg, Bg), 1)).astype(jnp.float32)
        pos = jnp.sum(sim * eye, axis=1, keepdims=True)
        tot = jnp.sum(sim, axis=1, keepdims=True)
        lvec = jnp.log(pos / (tot - pos))
        out_ref[...] = -jnp.mean(lvec) * jnp.ones((1, 1), jnp.float32)

    return pl.pallas_call(
        body,
        out_shape=jax.ShapeDtypeStruct((1, 1), jnp.float32),
    )(ch, gh, pp["W1"], pp["b1"].reshape(1, -1), pp["W2"], pp["b2"].reshape(1, -1))


def kernel(x, edge_index, graph_ids, mask_nodes, enc_params, con_params,
           proj_params, mask_token):
    N, D = x.shape
    src = edge_index[0].astype(jnp.int32)
    dst = edge_index[1].astype(jnp.int32)
    mask_nodes = mask_nodes.astype(jnp.int32)
    gid2 = graph_ids.astype(jnp.int32).reshape(N, 1)
    zrows = jnp.zeros(((N // _NS) // 8 * 8, D), jnp.bfloat16)

    E = src.shape[0]
    NW = _NC * _NS
    EP = E // NW
    CH = _CH
    NCH = -(-EP // CH)
    n_pad = NCH * CH - EP
    src_r = src.reshape(NW, EP)
    dst_r = dst.reshape(NW, EP)
    if n_pad:
        pad_s = jnp.zeros((NW, n_pad), jnp.int32)
        pad_d = jnp.broadcast_to(N + jnp.arange(n_pad, dtype=jnp.int32),
                                 (NW, n_pad))
        src_r = jnp.concatenate([src_r, pad_s], axis=1)
        dst_r = jnp.concatenate([dst_r, pad_d], axis=1)
    srcp = src_r.reshape(NW * NCH * CH)
    dstp = dst_r.reshape(NW * NCH * CH)
    n_pad_rows = max(n_pad, 8)

    xm = _mask_apply(x, mask_nodes, mask_token)

    def encoder(h0, params):
        h = h0
        hb = _cast_bf16(h0)
        pools = []
        for p in params:
            agg2 = _edge_segsum(hb, srcp, dstp, zrows, n_pad_rows, NCH)
            h, hb, pool = _dense_layer(h, agg2, p, gid2)
            pools.append(pool)
        return h, jnp.concatenate(pools, axis=1)

    _, gh = encoder(xm, enc_params)
    _, ch = encoder(x, con_params)
    out = _head(ch, gh, proj_params)
    return out[0, 0]
